# Initial kernel scaffold; baseline (speedup 1.0000x reference)
import jax
import jax.numpy as jnp
from jax.experimental import pallas as pl


def kernel(x, edge_index, batch, W_in, b_in, W_c1, b_c1, W_c2, b_c2, W_out, b_out):
    raise NotImplementedError("write your pallas kernel here")


# speedup vs baseline: 14.4705x; 14.4705x over previous
import functools

import jax
import jax.numpy as jnp
from jax import lax
from jax.experimental import pallas as pl
from jax.experimental.pallas import tpu as pltpu
from jax.experimental.pallas import tpu_sc as plsc

N = 10000
E = 320000
D = 128
G = 64
NC = 2
NS = 16
NW = NC * NS
K = 128
EPT = 10112
CPT = EPT // K
EP = NW * EPT
NP = 10240
RPS = NP // NS
BK = 2048
GRID = NP // BK


@functools.cache
def _mesh():
    return plsc.VectorSubcoreMesh(
        core_axis_name="c", subcore_axis_name="s", num_cores=NC, num_subcores=NS
    )


@functools.cache
def _sc_degree_kernel():
    return pl.kernel(
        _sc_degree,
        out_type=jax.ShapeDtypeStruct((NC, NP, 16), jnp.float32),
        mesh=_mesh(),
        scratch_types=[
            pltpu.VMEM((CPT, K), jnp.int32),
            pltpu.VMEM((K, 16), jnp.float32),
            pltpu.VMEM((K, 16), jnp.float32),
            pltpu.MemorySpace.VMEM_SHARED((NP, 16), jnp.float32),
        ],
    )


def _sc_degree(dst_hbm, out_hbm, dst_v, ones_v, zeros_v, acc):
    c = lax.axis_index("c")
    s = lax.axis_index("s")
    wid = c * NS + s
    pltpu.sync_copy(dst_hbm.at[wid], dst_v)

    def fill(r, carry):
        ones_v[r, :] = jnp.full((16,), 1.0, jnp.float32)
        zeros_v[r, :] = jnp.zeros((16,), jnp.float32)
        return carry

    lax.fori_loop(0, K, fill, 0)
    for k in range(RPS // K):
        pltpu.sync_copy(zeros_v, acc.at[pl.ds(s * RPS + k * K, K)])
    plsc.subcore_barrier()

    def body(i, carry):
        pltpu.sync_copy(ones_v, acc.at[dst_v.at[i]], add=True)
        return carry

    lax.fori_loop(0, CPT, body, 0)
    plsc.subcore_barrier()
    pltpu.sync_copy(acc.at[pl.ds(s * RPS, RPS)], out_hbm.at[c, pl.ds(s * RPS, RPS)])


@functools.cache
def _sc_aggregate_kernel():
    return pl.kernel(
        _sc_aggregate,
        out_type=jax.ShapeDtypeStruct((NC, NP, D), jnp.float32),
        mesh=_mesh(),
        scratch_types=[
            pltpu.VMEM((CPT, K), jnp.int32),
            pltpu.VMEM((CPT, K), jnp.int32),
            pltpu.VMEM((K, D), jnp.float32),
            pltpu.MemorySpace.VMEM_SHARED((NP, D), jnp.float32),
            pltpu.SemaphoreType.DMA,
        ],
    )


def _sc_aggregate(g_hbm, src_hbm, dst_hbm, out_hbm, src_v, dst_v, rows_v, acc, sem):
    c = lax.axis_index("c")
    s = lax.axis_index("s")
    wid = c * NS + s
    pltpu.sync_copy(src_hbm.at[wid], src_v)
    pltpu.sync_copy(dst_hbm.at[wid], dst_v)

    def zfill(r, carry):
        for j in range(D // 16):
            rows_v[r, pl.ds(j * 16, 16)] = jnp.zeros((16,), jnp.float32)
        return carry

    lax.fori_loop(0, K, zfill, 0)
    for k in range(RPS // K):
        pltpu.sync_copy(rows_v, acc.at[pl.ds(s * RPS + k * K, K)])
    plsc.subcore_barrier()

    def body(i, carry):
        pltpu.async_copy(g_hbm.at[src_v.at[i]], rows_v, sem).wait()
        pltpu.sync_copy(rows_v, acc.at[dst_v.at[i]], add=True)
        return carry

    lax.fori_loop(0, CPT, body, 0)
    plsc.subcore_barrier()
    pltpu.sync_copy(acc.at[pl.ds(s * RPS, RPS)], out_hbm.at[c, pl.ds(s * RPS, RPS)])


def _dinv_block(degp, step):
    deg = degp[0, :, 0:1] + degp[1, :, 0:1] + 1.0
    rid = step * BK + lax.broadcasted_iota(jnp.int32, (BK, 1), 0)
    return jnp.where(rid < N, lax.rsqrt(deg), 0.0)


def _lrelu(v):
    return jnp.where(v >= 0, v, 0.4 * v)


def _tc1_body(x_ref, degp_ref, win_ref, bin_ref, wc1_ref, g_ref):
    dinv = _dinv_block(degp_ref[...], pl.program_id(0))
    h = _lrelu(jnp.dot(x_ref[...], win_ref[...],
                       preferred_element_type=jnp.float32) + bin_ref[...])
    g_ref[...] = jnp.dot(h, wc1_ref[...],
                         preferred_element_type=jnp.float32) * dinv


def _tc2_body(p_ref, g_ref, degp_ref, b_ref, w_ref, o_ref):
    dinv = _dinv_block(degp_ref[...], pl.program_id(0))
    a = (p_ref[0] + p_ref[1] + g_ref[...]) * dinv + b_ref[...]
    h = _lrelu(a)
    o_ref[...] = jnp.dot(h, w_ref[...],
                         preferred_element_type=jnp.float32) * dinv


def _tc3_body(p_ref, g_ref, degp_ref, b_ref, batch_ref, wout_ref, bout_ref,
              o_ref, psum, cnt):
    i = pl.program_id(0)

    @pl.when(i == 0)
    def _():
        psum[...] = jnp.zeros_like(psum)
        cnt[...] = jnp.zeros_like(cnt)

    dinv = _dinv_block(degp_ref[...], i)
    h = _lrelu((p_ref[0] + p_ref[1] + g_ref[...]) * dinv + b_ref[...])
    onehot = (batch_ref[...] == lax.broadcasted_iota(jnp.int32, (1, G), 1)
              ).astype(jnp.float32)
    psum[...] += lax.dot_general(onehot, h, (((0,), (0,)), ((), ())),
                                 preferred_element_type=jnp.float32)
    cnt[...] += lax.dot_general(onehot, jnp.ones((BK, 1), jnp.float32),
                                (((0,), (0,)), ((), ())),
                                preferred_element_type=jnp.float32)

    @pl.when(i == GRID - 1)
    def _():
        pooled = psum[...] / jnp.maximum(cnt[...], 1.0)
        o_ref[...] = jnp.dot(pooled, wout_ref[...],
                             preferred_element_type=jnp.float32) + bout_ref[...]


def _row_spec(i_map=lambda i: (i, 0)):
    return pl.BlockSpec((BK, D), i_map)


_FULL = lambda shape: pl.BlockSpec(shape, lambda i: tuple(0 for _ in shape))
_DEGP_SPEC = pl.BlockSpec((NC, BK, 16), lambda i: (0, i, 0))
_PART_SPEC = pl.BlockSpec((NC, BK, D), lambda i: (0, i, 0))


def _tc1(x_p, degp, W_in, b_in, W_c1):
    return pl.pallas_call(
        _tc1_body,
        grid=(GRID,),
        in_specs=[_row_spec(), _DEGP_SPEC, _FULL((D, D)), _FULL((1, D)),
                  _FULL((D, D))],
        out_specs=_row_spec(),
        out_shape=jax.ShapeDtypeStruct((NP, D), jnp.float32),
    )(x_p, degp, W_in, b_in[None, :], W_c1)


def _tc2(part, g, degp, b, W):
    return pl.pallas_call(
        _tc2_body,
        grid=(GRID,),
        in_specs=[_PART_SPEC, _row_spec(), _DEGP_SPEC, _FULL((1, D)),
                  _FULL((D, D))],
        out_specs=_row_spec(),
        out_shape=jax.ShapeDtypeStruct((NP, D), jnp.float32),
    )(part, g, degp, b[None, :], W)


def _tc3(part, g, degp, b, batch_p, W_out, b_out):
    return pl.pallas_call(
        _tc3_body,
        grid=(GRID,),
        in_specs=[_PART_SPEC, _row_spec(), _DEGP_SPEC, _FULL((1, D)),
                  pl.BlockSpec((BK, 1), lambda i: (i, 0)), _FULL((D, D)),
                  _FULL((1, D))],
        out_specs=_FULL((G, D)),
        out_shape=jax.ShapeDtypeStruct((G, D), jnp.float32),
        scratch_shapes=[pltpu.VMEM((G, D), jnp.float32),
                        pltpu.VMEM((G, 1), jnp.float32)],
        compiler_params=pltpu.CompilerParams(
            dimension_semantics=("arbitrary",)),
    )(part, g, degp, b[None, :], batch_p, W_out, b_out[None, :])


def kernel(x, edge_index, batch, W_in, b_in, W_c1, b_c1, W_c2, b_c2, W_out, b_out):
    padv = jnp.full((EP - E,), N, jnp.int32)
    src_r = jnp.concatenate([edge_index[0], padv]).reshape(NW, CPT, K)
    dst_r = jnp.concatenate([edge_index[1], padv]).reshape(NW, CPT, K)
    x_p = jnp.pad(x, ((0, NP - N), (0, 0)))
    batch_p = jnp.pad(batch, (0, NP - N), constant_values=G).reshape(NP, 1)

    degp = _sc_degree_kernel()(dst_r)
    g1 = _tc1(x_p, degp, W_in, b_in, W_c1)
    p1 = _sc_aggregate_kernel()(g1, src_r, dst_r)
    g2 = _tc2(p1, g1, degp, b_c1, W_c2)
    p2 = _sc_aggregate_kernel()(g2, src_r, dst_r)
    return _tc3(p2, g2, degp, b_c2, batch_p, W_out, b_out)
